# initial kernel scaffold (unmeasured)
import jax
import jax.numpy as jnp
from jax import lax
from jax.experimental import pallas as pl
from jax.experimental.pallas import tpu as pltpu

N_DEV = 4
M, N = 8192, 4096
CH = M // N_DEV
TR = 512

_CompilerParams = getattr(pltpu, "CompilerParams", None) or getattr(
    pltpu, "TPUCompilerParams"
)


def _allreduce(partial):
    def body(p_ref, out_ref, acc_a, acc_b, copy_sems, send_sems, recv_sems):
        d = lax.axis_index("i")
        left = (d - 1 + N_DEV) % N_DEV
        right = (d + 1) % N_DEV

        barrier_sem = pltpu.get_barrier_semaphore()
        for nbr in (left, right):
            pl.semaphore_signal(
                barrier_sem,
                inc=1,
                device_id=(nbr,),
                device_id_type=pl.DeviceIdType.MESH,
            )
        pl.semaphore_wait(barrier_sem, 2)

        def accum(c):
            for t in range(CH // TR):
                r0 = c * CH + t * TR
                ca = pltpu.make_async_copy(
                    out_ref.at[pl.ds(r0, TR), :], acc_a, copy_sems.at[0]
                )
                cb = pltpu.make_async_copy(
                    p_ref.at[pl.ds(r0, TR), :], acc_b, copy_sems.at[1]
                )
                ca.start()
                cb.start()
                ca.wait()
                cb.wait()
                acc_a[...] = acc_a[...] + acc_b[...]
                cw = pltpu.make_async_copy(
                    acc_a, out_ref.at[pl.ds(r0, TR), :], copy_sems.at[0]
                )
                cw.start()
                cw.wait()

        for s in range(N_DEV - 1):
            c_send = (d - s + N_DEV) % N_DEV
            c_recv = (d - s - 1 + N_DEV) % N_DEV
            src = p_ref if s == 0 else out_ref
            rdma = pltpu.make_async_remote_copy(
                src_ref=src.at[pl.ds(c_send * CH, CH), :],
                dst_ref=out_ref.at[pl.ds(c_send * CH, CH), :],
                send_sem=send_sems.at[s],
                recv_sem=recv_sems.at[s],
                device_id=(right,),
                device_id_type=pl.DeviceIdType.MESH,
            )
            rdma.start()
            rdma.wait()
            accum(c_recv)

        for a in range(N_DEV - 1):
            c_send = (d + 1 - a + N_DEV) % N_DEV
            rdma = pltpu.make_async_remote_copy(
                src_ref=out_ref.at[pl.ds(c_send * CH, CH), :],
                dst_ref=out_ref.at[pl.ds(c_send * CH, CH), :],
                send_sem=send_sems.at[N_DEV - 1 + a],
                recv_sem=recv_sems.at[N_DEV - 1 + a],
                device_id=(right,),
                device_id_type=pl.DeviceIdType.MESH,
            )
            rdma.start()
            rdma.wait()

    return pl.pallas_call(
        body,
        out_shape=jax.ShapeDtypeStruct((M, N), jnp.float32),
        in_specs=[pl.BlockSpec(memory_space=pltpu.ANY)],
        out_specs=pl.BlockSpec(memory_space=pltpu.ANY),
        scratch_shapes=[
            pltpu.VMEM((TR, N), jnp.float32),
            pltpu.VMEM((TR, N), jnp.float32),
            pltpu.SemaphoreType.DMA((2,)),
            pltpu.SemaphoreType.DMA((2 * (N_DEV - 1),)),
            pltpu.SemaphoreType.DMA((2 * (N_DEV - 1),)),
        ],
        compiler_params=_CompilerParams(collective_id=0),
    )(partial)


def kernel(x, w_mat):
    partial = jnp.dot(x, w_mat, preferred_element_type=jnp.float32)
    return _allreduce(partial)


# baseline (device time: 2570122 ns/iter reference)
import jax
import jax.numpy as jnp
from jax import lax
from jax.experimental import pallas as pl
from jax.experimental.pallas import tpu as pltpu

N_DEV = 4
M, N = 8192, 4096
CH = M // N_DEV
TR = 512

_CompilerParams = getattr(pltpu, "CompilerParams", None) or getattr(
    pltpu, "TPUCompilerParams"
)


def _allreduce(partial):
    def body(p_ref, out_ref, acc_a, acc_b, copy_sems, send_sems, recv_sems):
        d = lax.axis_index("i")
        left = (d - 1 + N_DEV) % N_DEV
        right = (d + 1) % N_DEV

        barrier_sem = pltpu.get_barrier_semaphore()
        for nbr in (left, right):
            pl.semaphore_signal(
                barrier_sem,
                inc=1,
                device_id=(nbr,),
                device_id_type=pl.DeviceIdType.MESH,
            )
        pl.semaphore_wait(barrier_sem, 2)

        def accum(c):
            for t in range(CH // TR):
                r0 = c * CH + t * TR
                ca = pltpu.make_async_copy(
                    out_ref.at[pl.ds(r0, TR), :], acc_a, copy_sems.at[0]
                )
                cb = pltpu.make_async_copy(
                    p_ref.at[pl.ds(r0, TR), :], acc_b, copy_sems.at[1]
                )
                ca.start()
                cb.start()
                ca.wait()
                cb.wait()
                acc_a[...] = acc_a[...] + acc_b[...]
                cw = pltpu.make_async_copy(
                    acc_a, out_ref.at[pl.ds(r0, TR), :], copy_sems.at[0]
                )
                cw.start()
                cw.wait()

        for s in range(N_DEV - 1):
            c_send = (d - s + N_DEV) % N_DEV
            c_recv = (d - s - 1 + N_DEV) % N_DEV
            src = p_ref if s == 0 else out_ref
            rdma = pltpu.make_async_remote_copy(
                src_ref=src.at[pl.ds(c_send * CH, CH), :],
                dst_ref=out_ref.at[pl.ds(c_send * CH, CH), :],
                send_sem=send_sems.at[s],
                recv_sem=recv_sems.at[s],
                device_id=(right,),
                device_id_type=pl.DeviceIdType.MESH,
            )
            rdma.start()
            rdma.wait()
            accum(c_recv)

        for a in range(N_DEV - 1):
            c_send = (d + 1 - a + N_DEV) % N_DEV
            rdma = pltpu.make_async_remote_copy(
                src_ref=out_ref.at[pl.ds(c_send * CH, CH), :],
                dst_ref=out_ref.at[pl.ds(c_send * CH, CH), :],
                send_sem=send_sems.at[N_DEV - 1 + a],
                recv_sem=recv_sems.at[N_DEV - 1 + a],
                device_id=(right,),
                device_id_type=pl.DeviceIdType.MESH,
            )
            rdma.start()
            rdma.wait()

    return pl.pallas_call(
        body,
        out_shape=jax.ShapeDtypeStruct((M, N), jnp.float32),
        in_specs=[pl.BlockSpec(memory_space=pl.ANY)],
        out_specs=pl.BlockSpec(memory_space=pl.ANY),
        scratch_shapes=[
            pltpu.VMEM((TR, N), jnp.float32),
            pltpu.VMEM((TR, N), jnp.float32),
            pltpu.SemaphoreType.DMA((2,)),
            pltpu.SemaphoreType.DMA((2 * (N_DEV - 1),)),
            pltpu.SemaphoreType.DMA((2 * (N_DEV - 1),)),
        ],
        compiler_params=_CompilerParams(collective_id=0),
    )(partial)


def kernel(x, w_mat):
    partial = jnp.dot(x, w_mat, preferred_element_type=jnp.float32)
    return _allreduce(partial)


# device time: 1513323 ns/iter; 1.6983x vs baseline; 1.6983x over previous
import jax
import jax.numpy as jnp
from jax import lax
from jax.experimental import pallas as pl
from jax.experimental.pallas import tpu as pltpu

N_DEV = 4
M, N = 8192, 4096
CH = M // N_DEV
HALF = N // 2
TR = 512

_CompilerParams = getattr(pltpu, "CompilerParams", None) or getattr(
    pltpu, "TPUCompilerParams"
)


def _allreduce(partial):
    def body(p_ref, out_ref, acc_a, acc_b, copy_sems, send_sems, recv_sems):
        d = lax.axis_index("i")
        left = (d - 1 + N_DEV) % N_DEV
        right = (d + 1) % N_DEV

        barrier_sem = pltpu.get_barrier_semaphore()
        for nbr in (left, right):
            pl.semaphore_signal(
                barrier_sem,
                inc=1,
                device_id=(nbr,),
                device_id_type=pl.DeviceIdType.MESH,
            )
        pl.semaphore_wait(barrier_sem, 2)

        def accum_half(c, col0):
            for t in range(CH // TR):
                r0 = c * CH + t * TR
                ca = pltpu.make_async_copy(
                    out_ref.at[pl.ds(r0, TR), pl.ds(col0, HALF)],
                    acc_a,
                    copy_sems.at[0],
                )
                cb = pltpu.make_async_copy(
                    p_ref.at[pl.ds(r0, TR), pl.ds(col0, HALF)],
                    acc_b,
                    copy_sems.at[1],
                )
                ca.start()
                cb.start()
                ca.wait()
                cb.wait()
                acc_a[...] = acc_a[...] + acc_b[...]
                cw = pltpu.make_async_copy(
                    acc_a,
                    out_ref.at[pl.ds(r0, TR), pl.ds(col0, HALF)],
                    copy_sems.at[0],
                )
                cw.start()
                cw.wait()

        def ring_rdma(src, c_send, col0, sem_dir, step, target):
            return pltpu.make_async_remote_copy(
                src_ref=src.at[pl.ds(c_send * CH, CH), pl.ds(col0, HALF)],
                dst_ref=out_ref.at[pl.ds(c_send * CH, CH), pl.ds(col0, HALF)],
                send_sem=send_sems.at[sem_dir, step],
                recv_sem=recv_sems.at[sem_dir, step],
                device_id=(target,),
                device_id_type=pl.DeviceIdType.MESH,
            )

        for s in range(N_DEV - 1):
            c_sr = (d - s + N_DEV) % N_DEV
            c_rr = (d - s - 1 + N_DEV) % N_DEV
            c_sl = (d + s) % N_DEV
            c_rl = (d + s + 1) % N_DEV
            src = p_ref if s == 0 else out_ref
            r_rdma = ring_rdma(src, c_sr, 0, 0, s, right)
            l_rdma = ring_rdma(src, c_sl, HALF, 1, s, left)
            r_rdma.start()
            l_rdma.start()
            r_rdma.wait()
            l_rdma.wait()
            accum_half(c_rr, 0)
            accum_half(c_rl, HALF)

        for a in range(N_DEV - 1):
            c_ar = (d + 1 - a + N_DEV) % N_DEV
            c_al = (d - 1 + a + N_DEV) % N_DEV
            r_rdma = ring_rdma(out_ref, c_ar, 0, 0, N_DEV - 1 + a, right)
            l_rdma = ring_rdma(out_ref, c_al, HALF, 1, N_DEV - 1 + a, left)
            r_rdma.start()
            l_rdma.start()
            r_rdma.wait()
            l_rdma.wait()

    return pl.pallas_call(
        body,
        out_shape=jax.ShapeDtypeStruct((M, N), jnp.float32),
        in_specs=[pl.BlockSpec(memory_space=pl.ANY)],
        out_specs=pl.BlockSpec(memory_space=pl.ANY),
        scratch_shapes=[
            pltpu.VMEM((TR, HALF), jnp.float32),
            pltpu.VMEM((TR, HALF), jnp.float32),
            pltpu.SemaphoreType.DMA((2,)),
            pltpu.SemaphoreType.DMA((2, 2 * (N_DEV - 1))),
            pltpu.SemaphoreType.DMA((2, 2 * (N_DEV - 1))),
        ],
        compiler_params=_CompilerParams(collective_id=0),
    )(partial)


def kernel(x, w_mat):
    partial = jnp.dot(x, w_mat, preferred_element_type=jnp.float32)
    return _allreduce(partial)


# device time: 1407635 ns/iter; 1.8258x vs baseline; 1.0751x over previous
import jax
import jax.numpy as jnp
from jax import lax
from jax.experimental import pallas as pl
from jax.experimental.pallas import tpu as pltpu

N_DEV = 4
M, N = 8192, 4096
CH = M // N_DEV
HALF = N // 2
SUB = 4
SR = CH // SUB

_CompilerParams = getattr(pltpu, "CompilerParams", None) or getattr(
    pltpu, "TPUCompilerParams"
)


def _allreduce(partial):
    def body(p_ref, out_ref, acc_a, acc_b, copy_sems, send_sems, recv_sems):
        d = lax.axis_index("i")
        left = (d - 1 + N_DEV) % N_DEV
        right = (d + 1) % N_DEV

        barrier_sem = pltpu.get_barrier_semaphore()
        for nbr in (left, right):
            pl.semaphore_signal(
                barrier_sem,
                inc=1,
                device_id=(nbr,),
                device_id_type=pl.DeviceIdType.MESH,
            )
        pl.semaphore_wait(barrier_sem, 2)

        def accum_sub(c, u, col0):
            r0 = c * CH + u * SR
            ca = pltpu.make_async_copy(
                out_ref.at[pl.ds(r0, SR), pl.ds(col0, HALF)],
                acc_a,
                copy_sems.at[0],
            )
            cb = pltpu.make_async_copy(
                p_ref.at[pl.ds(r0, SR), pl.ds(col0, HALF)],
                acc_b,
                copy_sems.at[1],
            )
            ca.start()
            cb.start()
            ca.wait()
            cb.wait()
            acc_a[...] = acc_a[...] + acc_b[...]
            cw = pltpu.make_async_copy(
                acc_a,
                out_ref.at[pl.ds(r0, SR), pl.ds(col0, HALF)],
                copy_sems.at[0],
            )
            cw.start()
            cw.wait()

        def sub_rdma(src, c_send, u, col0, sem_dir, step, target):
            return pltpu.make_async_remote_copy(
                src_ref=src.at[pl.ds(c_send * CH + u * SR, SR), pl.ds(col0, HALF)],
                dst_ref=out_ref.at[pl.ds(c_send * CH + u * SR, SR), pl.ds(col0, HALF)],
                send_sem=send_sems.at[sem_dir, step, u],
                recv_sem=recv_sems.at[sem_dir, step, u],
                device_id=(target,),
                device_id_type=pl.DeviceIdType.MESH,
            )

        def ring_rdma(src, c_send, col0, sem_dir, step, target):
            return pltpu.make_async_remote_copy(
                src_ref=src.at[pl.ds(c_send * CH, CH), pl.ds(col0, HALF)],
                dst_ref=out_ref.at[pl.ds(c_send * CH, CH), pl.ds(col0, HALF)],
                send_sem=send_sems.at[sem_dir, step, 0],
                recv_sem=recv_sems.at[sem_dir, step, 0],
                device_id=(target,),
                device_id_type=pl.DeviceIdType.MESH,
            )

        for s in range(N_DEV - 1):
            c_sr = (d - s + N_DEV) % N_DEV
            c_rr = (d - s - 1 + N_DEV) % N_DEV
            c_sl = (d + s) % N_DEV
            c_rl = (d + s + 1) % N_DEV
            src = p_ref if s == 0 else out_ref
            rdmas = []
            for u in range(SUB):
                r_rdma = sub_rdma(src, c_sr, u, 0, 0, s, right)
                l_rdma = sub_rdma(src, c_sl, u, HALF, 1, s, left)
                r_rdma.start()
                l_rdma.start()
                rdmas.append((r_rdma, l_rdma))
            for u in range(SUB):
                r_rdma, l_rdma = rdmas[u]
                r_rdma.wait_recv()
                accum_sub(c_rr, u, 0)
                l_rdma.wait_recv()
                accum_sub(c_rl, u, HALF)
            for u in range(SUB):
                rdmas[u][0].wait_send()
                rdmas[u][1].wait_send()

        for a in range(N_DEV - 1):
            c_ar = (d + 1 - a + N_DEV) % N_DEV
            c_al = (d - 1 + a + N_DEV) % N_DEV
            r_rdma = ring_rdma(out_ref, c_ar, 0, 0, N_DEV - 1 + a, right)
            l_rdma = ring_rdma(out_ref, c_al, HALF, 1, N_DEV - 1 + a, left)
            r_rdma.start()
            l_rdma.start()
            r_rdma.wait()
            l_rdma.wait()

    return pl.pallas_call(
        body,
        out_shape=jax.ShapeDtypeStruct((M, N), jnp.float32),
        in_specs=[pl.BlockSpec(memory_space=pl.ANY)],
        out_specs=pl.BlockSpec(memory_space=pl.ANY),
        scratch_shapes=[
            pltpu.VMEM((SR, HALF), jnp.float32),
            pltpu.VMEM((SR, HALF), jnp.float32),
            pltpu.SemaphoreType.DMA((2,)),
            pltpu.SemaphoreType.DMA((2, 2 * (N_DEV - 1), SUB)),
            pltpu.SemaphoreType.DMA((2, 2 * (N_DEV - 1), SUB)),
        ],
        compiler_params=_CompilerParams(collective_id=0),
    )(partial)


def kernel(x, w_mat):
    partial = jnp.dot(x, w_mat, preferred_element_type=jnp.float32)
    return _allreduce(partial)


# device time: 1247898 ns/iter; 2.0596x vs baseline; 1.1280x over previous
import jax
import jax.numpy as jnp
from jax import lax
from jax.experimental import pallas as pl
from jax.experimental.pallas import tpu as pltpu

N_DEV = 4
M, K, N = 8192, 2048, 4096
CH = M // N_DEV
HALF = N // 2
SUB = 8
SR = CH // SUB

_CompilerParams = getattr(pltpu, "CompilerParams", None) or getattr(
    pltpu, "TPUCompilerParams"
)


def kernel(x, w_mat):
    def body(
        x_ref,
        w_ref,
        out_ref,
        w_a,
        w_b,
        xb,
        st_a,
        st_b,
        acc_a,
        acc_b,
        copy_sems,
        send_sems,
        recv_sems,
        ag_send_sems,
        ag_recv_sems,
    ):
        d = lax.axis_index("i")
        left = (d - 1 + N_DEV) % N_DEV
        right = (d + 1) % N_DEV

        barrier_sem = pltpu.get_barrier_semaphore()
        for nbr in (left, right):
            pl.semaphore_signal(
                barrier_sem,
                inc=1,
                device_id=(nbr,),
                device_id_type=pl.DeviceIdType.MESH,
            )
        pl.semaphore_wait(barrier_sem, 2)

        cw0 = pltpu.make_async_copy(
            w_ref.at[:, pl.ds(0, HALF)], w_a, copy_sems.at[0]
        )
        cw1 = pltpu.make_async_copy(
            w_ref.at[:, pl.ds(HALF, HALF)], w_b, copy_sems.at[1]
        )
        cw0.start()
        cw1.start()
        cw0.wait()
        cw1.wait()

        def x_copy(c, u, slot):
            return pltpu.make_async_copy(
                x_ref.at[pl.ds(c * CH + u * SR, SR), :],
                xb.at[slot],
                copy_sems.at[slot],
            )

        def rs_rdma(src, rows0, col0, ring, step, u, target):
            return pltpu.make_async_remote_copy(
                src_ref=src,
                dst_ref=out_ref.at[pl.ds(rows0, SR), pl.ds(col0, HALF)],
                send_sem=send_sems.at[ring, step, u],
                recv_sem=recv_sems.at[ring, step, u],
                device_id=(target,),
                device_id_type=pl.DeviceIdType.MESH,
            )

        def hop0_pair(u, sl):
            r0 = d * CH + u * SR
            r = rs_rdma(st_a.at[sl], r0, 0, 0, 0, u, right)
            l = rs_rdma(st_b.at[sl], r0, HALF, 1, 0, u, left)
            return r, l

        x_copy(d, 0, 0).start()

        def hop0_body(u, carry):
            sl = u % 2

            @pl.when(u >= 2)
            def _():
                r_prev, l_prev = hop0_pair(u - 2, sl)
                r_prev.wait_send()
                l_prev.wait_send()

            x_copy(d, u, sl).wait()
            va = st_a.at[sl]
            va[...] = jnp.dot(xb[sl], w_a[...], preferred_element_type=jnp.float32)
            vb = st_b.at[sl]
            vb[...] = jnp.dot(xb[sl], w_b[...], preferred_element_type=jnp.float32)

            @pl.when(u + 1 < SUB)
            def _():
                x_copy(d, u + 1, 1 - sl).start()

            r, l = hop0_pair(u, sl)
            r.start()
            l.start()
            return carry

        lax.fori_loop(0, SUB, hop0_body, 0)

        for s in range(N_DEV - 1):
            c_rr = (d - s - 1 + N_DEV) % N_DEV
            c_rl = (d + s + 1) % N_DEV
            if s > 0:
                c_sr = (d - s + N_DEV) % N_DEV
                c_sl = (d + s) % N_DEV

                def send_body(u, carry, s=s, c_sr=c_sr, c_sl=c_sl):
                    rr0 = c_sr * CH + u * SR
                    lr0 = c_sl * CH + u * SR
                    r = rs_rdma(
                        out_ref.at[pl.ds(rr0, SR), pl.ds(0, HALF)],
                        rr0, 0, 0, s, u, right,
                    )
                    l = rs_rdma(
                        out_ref.at[pl.ds(lr0, SR), pl.ds(HALF, HALF)],
                        lr0, HALF, 1, s, u, left,
                    )
                    r.start()
                    l.start()
                    return carry

                lax.fori_loop(0, SUB, send_body, 0)

            x_copy(c_rr, 0, 0).start()
            x_copy(c_rl, 0, 1).start()

            def accum_body(u, carry, s=s, c_rr=c_rr, c_rl=c_rl):
                sl = u % 2
                rr0 = c_rr * CH + u * SR
                lr0 = c_rl * CH + u * SR
                x_copy(c_rr, u, 0).wait()
                va = st_a.at[2 + sl]
                va[...] = jnp.dot(
                    xb[0], w_a[...], preferred_element_type=jnp.float32
                )
                x_copy(c_rl, u, 1).wait()
                vb = st_b.at[2 + sl]
                vb[...] = jnp.dot(
                    xb[1], w_b[...], preferred_element_type=jnp.float32
                )

                @pl.when(u + 1 < SUB)
                def _():
                    x_copy(c_rr, u + 1, 0).start()
                    x_copy(c_rl, u + 1, 1).start()

                rs_rdma(
                    out_ref.at[pl.ds(rr0, SR), pl.ds(0, HALF)],
                    rr0, 0, 0, s, u, right,
                ).wait_recv()
                la = pltpu.make_async_copy(
                    out_ref.at[pl.ds(rr0, SR), pl.ds(0, HALF)],
                    acc_a,
                    copy_sems.at[2],
                )
                la.start()
                rs_rdma(
                    out_ref.at[pl.ds(lr0, SR), pl.ds(HALF, HALF)],
                    lr0, HALF, 1, s, u, left,
                ).wait_recv()
                lb = pltpu.make_async_copy(
                    out_ref.at[pl.ds(lr0, SR), pl.ds(HALF, HALF)],
                    acc_b,
                    copy_sems.at[3],
                )
                lb.start()
                la.wait()
                acc_a[...] = acc_a[...] + st_a[2 + sl]
                sa = pltpu.make_async_copy(
                    acc_a,
                    out_ref.at[pl.ds(rr0, SR), pl.ds(0, HALF)],
                    copy_sems.at[4],
                )
                sa.start()
                lb.wait()
                acc_b[...] = acc_b[...] + st_b[2 + sl]
                sb = pltpu.make_async_copy(
                    acc_b,
                    out_ref.at[pl.ds(lr0, SR), pl.ds(HALF, HALF)],
                    copy_sems.at[5],
                )
                sb.start()
                sa.wait()
                sb.wait()
                return carry

            lax.fori_loop(0, SUB, accum_body, 0)

            if s == 0:
                for u in (SUB - 2, SUB - 1):
                    r, l = hop0_pair(u, u % 2)
                    r.wait_send()
                    l.wait_send()
            else:

                def wait_send_body(u, carry, s=s, c_sr=c_sr, c_sl=c_sl):
                    rr0 = c_sr * CH + u * SR
                    lr0 = c_sl * CH + u * SR
                    rs_rdma(
                        out_ref.at[pl.ds(rr0, SR), pl.ds(0, HALF)],
                        rr0, 0, 0, s, u, right,
                    ).wait_send()
                    rs_rdma(
                        out_ref.at[pl.ds(lr0, SR), pl.ds(HALF, HALF)],
                        lr0, HALF, 1, s, u, left,
                    ).wait_send()
                    return carry

                lax.fori_loop(0, SUB, wait_send_body, 0)

        for a in range(N_DEV - 1):
            c_ar = (d + 1 - a + N_DEV) % N_DEV
            c_al = (d - 1 + a + N_DEV) % N_DEV
            r = pltpu.make_async_remote_copy(
                src_ref=out_ref.at[pl.ds(c_ar * CH, CH), pl.ds(0, HALF)],
                dst_ref=out_ref.at[pl.ds(c_ar * CH, CH), pl.ds(0, HALF)],
                send_sem=ag_send_sems.at[0, a],
                recv_sem=ag_recv_sems.at[0, a],
                device_id=(right,),
                device_id_type=pl.DeviceIdType.MESH,
            )
            l = pltpu.make_async_remote_copy(
                src_ref=out_ref.at[pl.ds(c_al * CH, CH), pl.ds(HALF, HALF)],
                dst_ref=out_ref.at[pl.ds(c_al * CH, CH), pl.ds(HALF, HALF)],
                send_sem=ag_send_sems.at[1, a],
                recv_sem=ag_recv_sems.at[1, a],
                device_id=(left,),
                device_id_type=pl.DeviceIdType.MESH,
            )
            r.start()
            l.start()
            r.wait()
            l.wait()

    return pl.pallas_call(
        body,
        out_shape=jax.ShapeDtypeStruct((M, N), jnp.float32),
        in_specs=[
            pl.BlockSpec(memory_space=pl.ANY),
            pl.BlockSpec(memory_space=pl.ANY),
        ],
        out_specs=pl.BlockSpec(memory_space=pl.ANY),
        scratch_shapes=[
            pltpu.VMEM((K, HALF), jnp.float32),
            pltpu.VMEM((K, HALF), jnp.float32),
            pltpu.VMEM((2, SR, K), jnp.float32),
            pltpu.VMEM((4, SR, HALF), jnp.float32),
            pltpu.VMEM((4, SR, HALF), jnp.float32),
            pltpu.VMEM((SR, HALF), jnp.float32),
            pltpu.VMEM((SR, HALF), jnp.float32),
            pltpu.SemaphoreType.DMA((6,)),
            pltpu.SemaphoreType.DMA((2, N_DEV - 1, SUB)),
            pltpu.SemaphoreType.DMA((2, N_DEV - 1, SUB)),
            pltpu.SemaphoreType.DMA((2, N_DEV - 1)),
            pltpu.SemaphoreType.DMA((2, N_DEV - 1)),
        ],
        compiler_params=_CompilerParams(
            collective_id=0,
            vmem_limit_bytes=63 * 1024 * 1024,
        ),
    )(x, w_mat)


# device time: 1198498 ns/iter; 2.1445x vs baseline; 1.0412x over previous
import jax
import jax.numpy as jnp
from jax import lax
from jax.experimental import pallas as pl
from jax.experimental.pallas import tpu as pltpu

N_DEV = 4
M, K, N = 8192, 2048, 4096
CH = M // N_DEV
HALF = N // 2
SUB = 8
SR = CH // SUB

_CompilerParams = getattr(pltpu, "CompilerParams", None) or getattr(
    pltpu, "TPUCompilerParams"
)


def kernel(x, w_mat):
    def body(
        x_ref,
        w_ref,
        out_ref,
        w_a,
        w_b,
        xb,
        st_a,
        st_b,
        acc_a,
        acc_b,
        copy_sems,
        send_sems,
        recv_sems,
        ag_send_sems,
        ag_recv_sems,
    ):
        d = lax.axis_index("i")
        left = (d - 1 + N_DEV) % N_DEV
        right = (d + 1) % N_DEV

        barrier_sem = pltpu.get_barrier_semaphore()
        for nbr in (left, right):
            pl.semaphore_signal(
                barrier_sem,
                inc=1,
                device_id=(nbr,),
                device_id_type=pl.DeviceIdType.MESH,
            )
        pl.semaphore_wait(barrier_sem, 2)

        cw0 = pltpu.make_async_copy(
            w_ref.at[:, pl.ds(0, HALF)], w_a, copy_sems.at[0]
        )
        cw1 = pltpu.make_async_copy(
            w_ref.at[:, pl.ds(HALF, HALF)], w_b, copy_sems.at[1]
        )
        cw0.start()
        cw1.start()
        cw0.wait()
        cw1.wait()

        def x_copy(c, u, slot):
            return pltpu.make_async_copy(
                x_ref.at[pl.ds(c * CH + u * SR, SR), :],
                xb.at[slot],
                copy_sems.at[slot],
            )

        def rs_rdma(src, rows0, col0, ring, step, u, target):
            return pltpu.make_async_remote_copy(
                src_ref=src,
                dst_ref=out_ref.at[pl.ds(rows0, SR), pl.ds(col0, HALF)],
                send_sem=send_sems.at[ring, step, u],
                recv_sem=recv_sems.at[ring, step, u],
                device_id=(target,),
                device_id_type=pl.DeviceIdType.MESH,
            )

        def ag_rdma(c, u, ring, hop, target, col0):
            rows0 = c * CH + u * SR
            return pltpu.make_async_remote_copy(
                src_ref=out_ref.at[pl.ds(rows0, SR), pl.ds(col0, HALF)],
                dst_ref=out_ref.at[pl.ds(rows0, SR), pl.ds(col0, HALF)],
                send_sem=ag_send_sems.at[ring, hop, u],
                recv_sem=ag_recv_sems.at[ring, hop, u],
                device_id=(target,),
                device_id_type=pl.DeviceIdType.MESH,
            )

        def hop0_pair(u, sl):
            r0 = d * CH + u * SR
            r = rs_rdma(st_a.at[sl], r0, 0, 0, 0, u, right)
            l = rs_rdma(st_b.at[sl], r0, HALF, 1, 0, u, left)
            return r, l

        x_copy(d, 0, 0).start()

        def hop0_body(u, carry):
            sl = u % 2

            @pl.when(u >= 2)
            def _():
                r_prev, l_prev = hop0_pair(u - 2, sl)
                r_prev.wait_send()
                l_prev.wait_send()

            x_copy(d, u, sl).wait()
            va = st_a.at[sl]
            va[...] = jnp.dot(xb[sl], w_a[...], preferred_element_type=jnp.float32)
            vb = st_b.at[sl]
            vb[...] = jnp.dot(xb[sl], w_b[...], preferred_element_type=jnp.float32)

            @pl.when(u + 1 < SUB)
            def _():
                x_copy(d, u + 1, 1 - sl).start()

            r, l = hop0_pair(u, sl)
            r.start()
            l.start()
            return carry

        lax.fori_loop(0, SUB, hop0_body, 0)

        for s in range(N_DEV - 1):
            c_rr = (d - s - 1 + N_DEV) % N_DEV
            c_rl = (d + s + 1) % N_DEV
            x_copy(c_rr, 0, 0).start()
            x_copy(c_rl, 0, 1).start()

            def accum_body(u, carry, s=s, c_rr=c_rr, c_rl=c_rl):
                sl = u % 2
                rr0 = c_rr * CH + u * SR
                lr0 = c_rl * CH + u * SR
                x_copy(c_rr, u, 0).wait()
                va = st_a.at[2 + sl]
                va[...] = jnp.dot(
                    xb[0], w_a[...], preferred_element_type=jnp.float32
                )
                x_copy(c_rl, u, 1).wait()
                vb = st_b.at[2 + sl]
                vb[...] = jnp.dot(
                    xb[1], w_b[...], preferred_element_type=jnp.float32
                )

                @pl.when(u + 1 < SUB)
                def _():
                    x_copy(c_rr, u + 1, 0).start()
                    x_copy(c_rl, u + 1, 1).start()

                rs_rdma(
                    out_ref.at[pl.ds(rr0, SR), pl.ds(0, HALF)],
                    rr0, 0, 0, s, u, right,
                ).wait_recv()
                la = pltpu.make_async_copy(
                    out_ref.at[pl.ds(rr0, SR), pl.ds(0, HALF)],
                    acc_a,
                    copy_sems.at[2],
                )
                la.start()
                rs_rdma(
                    out_ref.at[pl.ds(lr0, SR), pl.ds(HALF, HALF)],
                    lr0, HALF, 1, s, u, left,
                ).wait_recv()
                lb = pltpu.make_async_copy(
                    out_ref.at[pl.ds(lr0, SR), pl.ds(HALF, HALF)],
                    acc_b,
                    copy_sems.at[3],
                )
                lb.start()
                la.wait()
                acc_a[...] = acc_a[...] + st_a[2 + sl]
                sa = pltpu.make_async_copy(
                    acc_a,
                    out_ref.at[pl.ds(rr0, SR), pl.ds(0, HALF)],
                    copy_sems.at[4],
                )
                sa.start()
                lb.wait()
                acc_b[...] = acc_b[...] + st_b[2 + sl]
                sb = pltpu.make_async_copy(
                    acc_b,
                    out_ref.at[pl.ds(lr0, SR), pl.ds(HALF, HALF)],
                    copy_sems.at[5],
                )
                sb.start()
                sa.wait()
                sb.wait()
                if s < N_DEV - 2:
                    rs_rdma(
                        out_ref.at[pl.ds(rr0, SR), pl.ds(0, HALF)],
                        rr0, 0, 0, s + 1, u, right,
                    ).start()
                    rs_rdma(
                        out_ref.at[pl.ds(lr0, SR), pl.ds(HALF, HALF)],
                        lr0, HALF, 1, s + 1, u, left,
                    ).start()
                return carry

            lax.fori_loop(0, SUB, accum_body, 0)

            if s == 0:
                for u in (SUB - 2, SUB - 1):
                    r, l = hop0_pair(u, u % 2)
                    r.wait_send()
                    l.wait_send()
            else:
                c_sr = (d - s + N_DEV) % N_DEV
                c_sl = (d + s) % N_DEV

                def wait_send_body(u, carry, s=s, c_sr=c_sr, c_sl=c_sl):
                    rr0 = c_sr * CH + u * SR
                    lr0 = c_sl * CH + u * SR
                    rs_rdma(
                        out_ref.at[pl.ds(rr0, SR), pl.ds(0, HALF)],
                        rr0, 0, 0, s, u, right,
                    ).wait_send()
                    rs_rdma(
                        out_ref.at[pl.ds(lr0, SR), pl.ds(HALF, HALF)],
                        lr0, HALF, 1, s, u, left,
                    ).wait_send()
                    return carry

                lax.fori_loop(0, SUB, wait_send_body, 0)

        for a in range(N_DEV - 1):
            c_ar = (d + 1 - a + N_DEV) % N_DEV
            c_al = (d - 1 + a + N_DEV) % N_DEV
            r = pltpu.make_async_remote_copy(
                src_ref=out_ref.at[pl.ds(c_ar * CH, CH), pl.ds(0, HALF)],
                dst_ref=out_ref.at[pl.ds(c_ar * CH, CH), pl.ds(0, HALF)],
                send_sem=ag_send_sems.at[0, a, 0],
                recv_sem=ag_recv_sems.at[0, a, 0],
                device_id=(right,),
                device_id_type=pl.DeviceIdType.MESH,
            )
            l = pltpu.make_async_remote_copy(
                src_ref=out_ref.at[pl.ds(c_al * CH, CH), pl.ds(HALF, HALF)],
                dst_ref=out_ref.at[pl.ds(c_al * CH, CH), pl.ds(HALF, HALF)],
                send_sem=ag_send_sems.at[1, a, 0],
                recv_sem=ag_recv_sems.at[1, a, 0],
                device_id=(left,),
                device_id_type=pl.DeviceIdType.MESH,
            )
            r.start()
            l.start()
            r.wait()
            l.wait()

    return pl.pallas_call(
        body,
        out_shape=jax.ShapeDtypeStruct((M, N), jnp.float32),
        in_specs=[
            pl.BlockSpec(memory_space=pl.ANY),
            pl.BlockSpec(memory_space=pl.ANY),
        ],
        out_specs=pl.BlockSpec(memory_space=pl.ANY),
        scratch_shapes=[
            pltpu.VMEM((K, HALF), jnp.float32),
            pltpu.VMEM((K, HALF), jnp.float32),
            pltpu.VMEM((2, SR, K), jnp.float32),
            pltpu.VMEM((4, SR, HALF), jnp.float32),
            pltpu.VMEM((4, SR, HALF), jnp.float32),
            pltpu.VMEM((SR, HALF), jnp.float32),
            pltpu.VMEM((SR, HALF), jnp.float32),
            pltpu.SemaphoreType.DMA((6,)),
            pltpu.SemaphoreType.DMA((2, N_DEV - 1, SUB)),
            pltpu.SemaphoreType.DMA((2, N_DEV - 1, SUB)),
            pltpu.SemaphoreType.DMA((2, N_DEV - 1, SUB)),
            pltpu.SemaphoreType.DMA((2, N_DEV - 1, SUB)),
        ],
        compiler_params=_CompilerParams(
            collective_id=0,
            vmem_limit_bytes=63 * 1024 * 1024,
        ),
    )(x, w_mat)


# device time: 1191288 ns/iter; 2.1574x vs baseline; 1.0061x over previous
import jax
import jax.numpy as jnp
from jax import lax
from jax.experimental import pallas as pl
from jax.experimental.pallas import tpu as pltpu

N_DEV = 4
M, K, N = 8192, 2048, 4096
CH = M // N_DEV
HALF = N // 2
SUB = 8
SR = CH // SUB

_CompilerParams = getattr(pltpu, "CompilerParams", None) or getattr(
    pltpu, "TPUCompilerParams"
)


def kernel(x, w_mat):
    def body(
        x_ref,
        w_ref,
        out_ref,
        w_a,
        w_b,
        xb,
        st_a,
        st_b,
        acc_a,
        acc_b,
        copy_sems,
        send_sems,
        recv_sems,
        ag_send_sems,
        ag_recv_sems,
    ):
        d = lax.axis_index("i")
        left = (d - 1 + N_DEV) % N_DEV
        right = (d + 1) % N_DEV

        cw0 = pltpu.make_async_copy(
            w_ref.at[:, pl.ds(0, HALF)], w_a, copy_sems.at[0]
        )
        cw1 = pltpu.make_async_copy(
            w_ref.at[:, pl.ds(HALF, HALF)], w_b, copy_sems.at[1]
        )
        cw0.start()
        cw1.start()

        barrier_sem = pltpu.get_barrier_semaphore()
        for nbr in (left, right):
            pl.semaphore_signal(
                barrier_sem,
                inc=1,
                device_id=(nbr,),
                device_id_type=pl.DeviceIdType.MESH,
            )
        pl.semaphore_wait(barrier_sem, 2)

        cw0.wait()
        cw1.wait()

        def x_copy(c, u, slot):
            return pltpu.make_async_copy(
                x_ref.at[pl.ds(c * CH + u * SR, SR), :],
                xb.at[slot],
                copy_sems.at[slot],
            )

        def rs_rdma(src, rows0, col0, ring, step, u, target):
            return pltpu.make_async_remote_copy(
                src_ref=src,
                dst_ref=out_ref.at[pl.ds(rows0, SR), pl.ds(col0, HALF)],
                send_sem=send_sems.at[ring, step, u],
                recv_sem=recv_sems.at[ring, step, u],
                device_id=(target,),
                device_id_type=pl.DeviceIdType.MESH,
            )

        def ag_rdma(c, u, ring, hop, target, col0):
            rows0 = c * CH + u * SR
            return pltpu.make_async_remote_copy(
                src_ref=out_ref.at[pl.ds(rows0, SR), pl.ds(col0, HALF)],
                dst_ref=out_ref.at[pl.ds(rows0, SR), pl.ds(col0, HALF)],
                send_sem=ag_send_sems.at[ring, hop, u],
                recv_sem=ag_recv_sems.at[ring, hop, u],
                device_id=(target,),
                device_id_type=pl.DeviceIdType.MESH,
            )

        def hop0_pair(u, sl):
            r0 = d * CH + u * SR
            r = rs_rdma(st_a.at[sl], r0, 0, 0, 0, u, right)
            l = rs_rdma(st_b.at[sl], r0, HALF, 1, 0, u, left)
            return r, l

        x_copy(d, 0, 0).start()

        def hop0_body(u, carry):
            sl = u % 2

            @pl.when(u >= 2)
            def _():
                r_prev, l_prev = hop0_pair(u - 2, sl)
                r_prev.wait_send()
                l_prev.wait_send()

            x_copy(d, u, sl).wait()
            va = st_a.at[sl]
            va[...] = jnp.dot(xb[sl], w_a[...], preferred_element_type=jnp.float32)
            vb = st_b.at[sl]
            vb[...] = jnp.dot(xb[sl], w_b[...], preferred_element_type=jnp.float32)

            @pl.when(u + 1 < SUB)
            def _():
                x_copy(d, u + 1, 1 - sl).start()

            r, l = hop0_pair(u, sl)
            r.start()
            l.start()
            return carry

        lax.fori_loop(0, SUB, hop0_body, 0)

        for s in range(N_DEV - 1):
            c_rr = (d - s - 1 + N_DEV) % N_DEV
            c_rl = (d + s + 1) % N_DEV
            x_copy(c_rr, 0, 0).start()
            x_copy(c_rl, 0, 1).start()

            def accum_body(u, carry, s=s, c_rr=c_rr, c_rl=c_rl):
                sl = u % 2
                rr0 = c_rr * CH + u * SR
                lr0 = c_rl * CH + u * SR
                x_copy(c_rr, u, 0).wait()
                va = st_a.at[2 + sl]
                va[...] = jnp.dot(
                    xb[0], w_a[...], preferred_element_type=jnp.float32
                )
                x_copy(c_rl, u, 1).wait()
                vb = st_b.at[2 + sl]
                vb[...] = jnp.dot(
                    xb[1], w_b[...], preferred_element_type=jnp.float32
                )

                @pl.when(u + 1 < SUB)
                def _():
                    x_copy(c_rr, u + 1, 0).start()
                    x_copy(c_rl, u + 1, 1).start()

                rs_rdma(
                    out_ref.at[pl.ds(rr0, SR), pl.ds(0, HALF)],
                    rr0, 0, 0, s, u, right,
                ).wait_recv()
                la = pltpu.make_async_copy(
                    out_ref.at[pl.ds(rr0, SR), pl.ds(0, HALF)],
                    acc_a,
                    copy_sems.at[2],
                )
                la.start()
                rs_rdma(
                    out_ref.at[pl.ds(lr0, SR), pl.ds(HALF, HALF)],
                    lr0, HALF, 1, s, u, left,
                ).wait_recv()
                lb = pltpu.make_async_copy(
                    out_ref.at[pl.ds(lr0, SR), pl.ds(HALF, HALF)],
                    acc_b,
                    copy_sems.at[3],
                )
                lb.start()
                la.wait()
                acc_a[...] = acc_a[...] + st_a[2 + sl]
                sa = pltpu.make_async_copy(
                    acc_a,
                    out_ref.at[pl.ds(rr0, SR), pl.ds(0, HALF)],
                    copy_sems.at[4],
                )
                sa.start()
                lb.wait()
                acc_b[...] = acc_b[...] + st_b[2 + sl]
                sb = pltpu.make_async_copy(
                    acc_b,
                    out_ref.at[pl.ds(lr0, SR), pl.ds(HALF, HALF)],
                    copy_sems.at[5],
                )
                sb.start()
                sa.wait()
                sb.wait()
                if s < N_DEV - 2:
                    rs_rdma(
                        out_ref.at[pl.ds(rr0, SR), pl.ds(0, HALF)],
                        rr0, 0, 0, s + 1, u, right,
                    ).start()
                    rs_rdma(
                        out_ref.at[pl.ds(lr0, SR), pl.ds(HALF, HALF)],
                        lr0, HALF, 1, s + 1, u, left,
                    ).start()
                else:
                    ag_rdma(c_rr, u, 0, 0, right, 0).start()
                    ag_rdma(c_rl, u, 1, 0, left, HALF).start()
                return carry

            lax.fori_loop(0, SUB, accum_body, 0)

            if s == 0:
                for u in (SUB - 2, SUB - 1):
                    r, l = hop0_pair(u, u % 2)
                    r.wait_send()
                    l.wait_send()
            else:
                c_sr = (d - s + N_DEV) % N_DEV
                c_sl = (d + s) % N_DEV

                def wait_send_body(u, carry, s=s, c_sr=c_sr, c_sl=c_sl):
                    rr0 = c_sr * CH + u * SR
                    lr0 = c_sl * CH + u * SR
                    rs_rdma(
                        out_ref.at[pl.ds(rr0, SR), pl.ds(0, HALF)],
                        rr0, 0, 0, s, u, right,
                    ).wait_send()
                    rs_rdma(
                        out_ref.at[pl.ds(lr0, SR), pl.ds(HALF, HALF)],
                        lr0, HALF, 1, s, u, left,
                    ).wait_send()
                    return carry

                lax.fori_loop(0, SUB, wait_send_body, 0)

        def ag0_recv_body(u, carry):
            ag_rdma(d, u, 0, 0, right, 0).wait_recv()
            ag_rdma(d, u, 1, 0, left, HALF).wait_recv()
            return carry

        lax.fori_loop(0, SUB, ag0_recv_body, 0)

        def ag0_ws_body(u, carry):
            ag_rdma((d + 1) % N_DEV, u, 0, 0, right, 0).wait_send()
            ag_rdma((d - 1 + N_DEV) % N_DEV, u, 1, 0, left, HALF).wait_send()
            return carry

        lax.fori_loop(0, SUB, ag0_ws_body, 0)

        for a in range(1, N_DEV - 1):
            c_ar = (d + 1 - a + N_DEV) % N_DEV
            c_al = (d - 1 + a + N_DEV) % N_DEV
            r = pltpu.make_async_remote_copy(
                src_ref=out_ref.at[pl.ds(c_ar * CH, CH), pl.ds(0, HALF)],
                dst_ref=out_ref.at[pl.ds(c_ar * CH, CH), pl.ds(0, HALF)],
                send_sem=ag_send_sems.at[0, a, 0],
                recv_sem=ag_recv_sems.at[0, a, 0],
                device_id=(right,),
                device_id_type=pl.DeviceIdType.MESH,
            )
            l = pltpu.make_async_remote_copy(
                src_ref=out_ref.at[pl.ds(c_al * CH, CH), pl.ds(HALF, HALF)],
                dst_ref=out_ref.at[pl.ds(c_al * CH, CH), pl.ds(HALF, HALF)],
                send_sem=ag_send_sems.at[1, a, 0],
                recv_sem=ag_recv_sems.at[1, a, 0],
                device_id=(left,),
                device_id_type=pl.DeviceIdType.MESH,
            )
            r.start()
            l.start()
            r.wait()
            l.wait()

    return pl.pallas_call(
        body,
        out_shape=jax.ShapeDtypeStruct((M, N), jnp.float32),
        in_specs=[
            pl.BlockSpec(memory_space=pl.ANY),
            pl.BlockSpec(memory_space=pl.ANY),
        ],
        out_specs=pl.BlockSpec(memory_space=pl.ANY),
        scratch_shapes=[
            pltpu.VMEM((K, HALF), jnp.float32),
            pltpu.VMEM((K, HALF), jnp.float32),
            pltpu.VMEM((2, SR, K), jnp.float32),
            pltpu.VMEM((4, SR, HALF), jnp.float32),
            pltpu.VMEM((4, SR, HALF), jnp.float32),
            pltpu.VMEM((SR, HALF), jnp.float32),
            pltpu.VMEM((SR, HALF), jnp.float32),
            pltpu.SemaphoreType.DMA((6,)),
            pltpu.SemaphoreType.DMA((2, N_DEV - 1, SUB)),
            pltpu.SemaphoreType.DMA((2, N_DEV - 1, SUB)),
            pltpu.SemaphoreType.DMA((2, N_DEV - 1, SUB)),
            pltpu.SemaphoreType.DMA((2, N_DEV - 1, SUB)),
        ],
        compiler_params=_CompilerParams(
            collective_id=0,
            vmem_limit_bytes=63 * 1024 * 1024,
        ),
    )(x, w_mat)


# device time: 1190173 ns/iter; 2.1595x vs baseline; 1.0009x over previous
import jax
import jax.numpy as jnp
from jax import lax
from jax.experimental import pallas as pl
from jax.experimental.pallas import tpu as pltpu

N_DEV = 4
M, K, N = 8192, 2048, 4096
CH = M // N_DEV
HALF = N // 2
SUB = 16
SR = CH // SUB

_CompilerParams = getattr(pltpu, "CompilerParams", None) or getattr(
    pltpu, "TPUCompilerParams"
)


def kernel(x, w_mat):
    def body(
        x_ref,
        w_ref,
        out_ref,
        w_a,
        w_b,
        xb,
        st_a,
        st_b,
        acc_a,
        acc_b,
        copy_sems,
        send_sems,
        recv_sems,
        ag_send_sems,
        ag_recv_sems,
    ):
        d = lax.axis_index("i")
        left = (d - 1 + N_DEV) % N_DEV
        right = (d + 1) % N_DEV

        cw0 = pltpu.make_async_copy(
            w_ref.at[:, pl.ds(0, HALF)], w_a, copy_sems.at[0]
        )
        cw1 = pltpu.make_async_copy(
            w_ref.at[:, pl.ds(HALF, HALF)], w_b, copy_sems.at[1]
        )
        cw0.start()
        cw1.start()

        barrier_sem = pltpu.get_barrier_semaphore()
        for nbr in (left, right):
            pl.semaphore_signal(
                barrier_sem,
                inc=1,
                device_id=(nbr,),
                device_id_type=pl.DeviceIdType.MESH,
            )
        pl.semaphore_wait(barrier_sem, 2)

        cw0.wait()
        cw1.wait()

        def x_copy(c, u, slot):
            return pltpu.make_async_copy(
                x_ref.at[pl.ds(c * CH + u * SR, SR), :],
                xb.at[slot],
                copy_sems.at[slot],
            )

        def rs_rdma(src, rows0, col0, ring, step, u, target):
            return pltpu.make_async_remote_copy(
                src_ref=src,
                dst_ref=out_ref.at[pl.ds(rows0, SR), pl.ds(col0, HALF)],
                send_sem=send_sems.at[ring, step, u],
                recv_sem=recv_sems.at[ring, step, u],
                device_id=(target,),
                device_id_type=pl.DeviceIdType.MESH,
            )

        def ag_rdma(c, u, ring, hop, target, col0):
            rows0 = c * CH + u * SR
            return pltpu.make_async_remote_copy(
                src_ref=out_ref.at[pl.ds(rows0, SR), pl.ds(col0, HALF)],
                dst_ref=out_ref.at[pl.ds(rows0, SR), pl.ds(col0, HALF)],
                send_sem=ag_send_sems.at[ring, hop, u],
                recv_sem=ag_recv_sems.at[ring, hop, u],
                device_id=(target,),
                device_id_type=pl.DeviceIdType.MESH,
            )

        def hop0_pair(u, sl):
            r0 = d * CH + u * SR
            r = rs_rdma(st_a.at[sl], r0, 0, 0, 0, u, right)
            l = rs_rdma(st_b.at[sl], r0, HALF, 1, 0, u, left)
            return r, l

        x_copy(d, 0, 0).start()

        def hop0_body(u, carry):
            sl = u % 2

            @pl.when(u >= 2)
            def _():
                r_prev, l_prev = hop0_pair(u - 2, sl)
                r_prev.wait_send()
                l_prev.wait_send()

            x_copy(d, u, sl).wait()
            va = st_a.at[sl]
            va[...] = jnp.dot(xb[sl], w_a[...], preferred_element_type=jnp.float32)
            vb = st_b.at[sl]
            vb[...] = jnp.dot(xb[sl], w_b[...], preferred_element_type=jnp.float32)

            @pl.when(u + 1 < SUB)
            def _():
                x_copy(d, u + 1, 1 - sl).start()

            r, l = hop0_pair(u, sl)
            r.start()
            l.start()
            return carry

        lax.fori_loop(0, SUB, hop0_body, 0)

        for s in range(N_DEV - 1):
            c_rr = (d - s - 1 + N_DEV) % N_DEV
            c_rl = (d + s + 1) % N_DEV
            x_copy(c_rr, 0, 0).start()
            x_copy(c_rl, 0, 1).start()

            def accum_body(u, carry, s=s, c_rr=c_rr, c_rl=c_rl):
                sl = u % 2
                rr0 = c_rr * CH + u * SR
                lr0 = c_rl * CH + u * SR
                x_copy(c_rr, u, 0).wait()
                va = st_a.at[2 + sl]
                va[...] = jnp.dot(
                    xb[0], w_a[...], preferred_element_type=jnp.float32
                )
                x_copy(c_rl, u, 1).wait()
                vb = st_b.at[2 + sl]
                vb[...] = jnp.dot(
                    xb[1], w_b[...], preferred_element_type=jnp.float32
                )

                @pl.when(u + 1 < SUB)
                def _():
                    x_copy(c_rr, u + 1, 0).start()
                    x_copy(c_rl, u + 1, 1).start()

                rs_rdma(
                    out_ref.at[pl.ds(rr0, SR), pl.ds(0, HALF)],
                    rr0, 0, 0, s, u, right,
                ).wait_recv()
                la = pltpu.make_async_copy(
                    out_ref.at[pl.ds(rr0, SR), pl.ds(0, HALF)],
                    acc_a,
                    copy_sems.at[2],
                )
                la.start()
                rs_rdma(
                    out_ref.at[pl.ds(lr0, SR), pl.ds(HALF, HALF)],
                    lr0, HALF, 1, s, u, left,
                ).wait_recv()
                lb = pltpu.make_async_copy(
                    out_ref.at[pl.ds(lr0, SR), pl.ds(HALF, HALF)],
                    acc_b,
                    copy_sems.at[3],
                )
                lb.start()
                la.wait()
                acc_a[...] = acc_a[...] + st_a[2 + sl]
                sa = pltpu.make_async_copy(
                    acc_a,
                    out_ref.at[pl.ds(rr0, SR), pl.ds(0, HALF)],
                    copy_sems.at[4],
                )
                sa.start()
                lb.wait()
                acc_b[...] = acc_b[...] + st_b[2 + sl]
                sb = pltpu.make_async_copy(
                    acc_b,
                    out_ref.at[pl.ds(lr0, SR), pl.ds(HALF, HALF)],
                    copy_sems.at[5],
                )
                sb.start()
                sa.wait()
                sb.wait()
                if s < N_DEV - 2:
                    rs_rdma(
                        out_ref.at[pl.ds(rr0, SR), pl.ds(0, HALF)],
                        rr0, 0, 0, s + 1, u, right,
                    ).start()
                    rs_rdma(
                        out_ref.at[pl.ds(lr0, SR), pl.ds(HALF, HALF)],
                        lr0, HALF, 1, s + 1, u, left,
                    ).start()
                else:
                    ag_rdma(c_rr, u, 0, 0, right, 0).start()
                    ag_rdma(c_rl, u, 1, 0, left, HALF).start()
                return carry

            lax.fori_loop(0, SUB, accum_body, 0)

            if s == 0:
                for u in (SUB - 2, SUB - 1):
                    r, l = hop0_pair(u, u % 2)
                    r.wait_send()
                    l.wait_send()
            else:
                c_sr = (d - s + N_DEV) % N_DEV
                c_sl = (d + s) % N_DEV

                def wait_send_body(u, carry, s=s, c_sr=c_sr, c_sl=c_sl):
                    rr0 = c_sr * CH + u * SR
                    lr0 = c_sl * CH + u * SR
                    rs_rdma(
                        out_ref.at[pl.ds(rr0, SR), pl.ds(0, HALF)],
                        rr0, 0, 0, s, u, right,
                    ).wait_send()
                    rs_rdma(
                        out_ref.at[pl.ds(lr0, SR), pl.ds(HALF, HALF)],
                        lr0, HALF, 1, s, u, left,
                    ).wait_send()
                    return carry

                lax.fori_loop(0, SUB, wait_send_body, 0)

        def ag0_recv_body(u, carry):
            ag_rdma(d, u, 0, 0, right, 0).wait_recv()
            ag_rdma(d, u, 1, 0, left, HALF).wait_recv()
            return carry

        lax.fori_loop(0, SUB, ag0_recv_body, 0)

        def ag0_ws_body(u, carry):
            ag_rdma((d + 1) % N_DEV, u, 0, 0, right, 0).wait_send()
            ag_rdma((d - 1 + N_DEV) % N_DEV, u, 1, 0, left, HALF).wait_send()
            return carry

        lax.fori_loop(0, SUB, ag0_ws_body, 0)

        for a in range(1, N_DEV - 1):
            c_ar = (d + 1 - a + N_DEV) % N_DEV
            c_al = (d - 1 + a + N_DEV) % N_DEV
            r = pltpu.make_async_remote_copy(
                src_ref=out_ref.at[pl.ds(c_ar * CH, CH), pl.ds(0, HALF)],
                dst_ref=out_ref.at[pl.ds(c_ar * CH, CH), pl.ds(0, HALF)],
                send_sem=ag_send_sems.at[0, a, 0],
                recv_sem=ag_recv_sems.at[0, a, 0],
                device_id=(right,),
                device_id_type=pl.DeviceIdType.MESH,
            )
            l = pltpu.make_async_remote_copy(
                src_ref=out_ref.at[pl.ds(c_al * CH, CH), pl.ds(HALF, HALF)],
                dst_ref=out_ref.at[pl.ds(c_al * CH, CH), pl.ds(HALF, HALF)],
                send_sem=ag_send_sems.at[1, a, 0],
                recv_sem=ag_recv_sems.at[1, a, 0],
                device_id=(left,),
                device_id_type=pl.DeviceIdType.MESH,
            )
            r.start()
            l.start()
            r.wait()
            l.wait()

    return pl.pallas_call(
        body,
        out_shape=jax.ShapeDtypeStruct((M, N), jnp.float32),
        in_specs=[
            pl.BlockSpec(memory_space=pl.ANY),
            pl.BlockSpec(memory_space=pl.ANY),
        ],
        out_specs=pl.BlockSpec(memory_space=pl.ANY),
        scratch_shapes=[
            pltpu.VMEM((K, HALF), jnp.float32),
            pltpu.VMEM((K, HALF), jnp.float32),
            pltpu.VMEM((2, SR, K), jnp.float32),
            pltpu.VMEM((4, SR, HALF), jnp.float32),
            pltpu.VMEM((4, SR, HALF), jnp.float32),
            pltpu.VMEM((SR, HALF), jnp.float32),
            pltpu.VMEM((SR, HALF), jnp.float32),
            pltpu.SemaphoreType.DMA((6,)),
            pltpu.SemaphoreType.DMA((2, N_DEV - 1, SUB)),
            pltpu.SemaphoreType.DMA((2, N_DEV - 1, SUB)),
            pltpu.SemaphoreType.DMA((2, N_DEV - 1, SUB)),
            pltpu.SemaphoreType.DMA((2, N_DEV - 1, SUB)),
        ],
        compiler_params=_CompilerParams(
            collective_id=0,
            vmem_limit_bytes=63 * 1024 * 1024,
        ),
    )(x, w_mat)
